# baseline (speedup 1.0000x reference)
import functools

import jax
import jax.numpy as jnp
from jax import lax
from jax.experimental import pallas as pl
from jax.experimental.pallas import tpu as pltpu
from jax.experimental.pallas import tpu_sc as plsc

V = 100000
D = 64
B = 4096
X_MAX = 100.0
ALPHA = 0.75

_NC = 2
_NS = 16
_NW = _NC * _NS
_BPW = B // _NW
_L = 16

_GATHER_DN = lax.GatherDimensionNumbers(
    offset_dims=(), collapsed_slice_dims=(0,), start_index_map=(0,))


def _lane_perm(x, idx):
    return lax.gather(x, idx.reshape(_L, 1), _GATHER_DN, (1,),
                      mode=lax.GatherScatterMode.PROMISE_IN_BOUNDS)


def _sc_gather_dot(fe_hbm, ce_hbm, fb_hbm, cb_hbm, fi_hbm, ci_hbm,
                   ep_hbm, bs_hbm,
                   fi_v, ci_v, fi2_v, ci2_v, fe_v, ce_v, fb_v, cb_v,
                   ep_v, bs_v,
                   sem0, sem1, sem2, sem3):
    wid = lax.axis_index("s") * _NC + lax.axis_index("c")
    base = wid * _BPW

    pltpu.sync_copy(fi_hbm.at[pl.ds(base, _BPW)], fi_v)
    pltpu.sync_copy(ci_hbm.at[pl.ds(base, _BPW)], ci_v)

    for g in range(_BPW // _L):
        sl = pl.ds(g * _L, _L)
        fi2_v[sl] = lax.shift_right_logical(fi_v[sl], 1)
        ci2_v[sl] = lax.shift_right_logical(ci_v[sl], 1)

    g0 = pltpu.async_copy(fe_hbm.at[fi2_v], fe_v, sem0)
    g1 = pltpu.async_copy(ce_hbm.at[ci2_v], ce_v, sem1)
    g2 = pltpu.async_copy(fb_hbm.at[fi_v], fb_v, sem2)
    g3 = pltpu.async_copy(cb_hbm.at[ci_v], cb_v, sem3)
    g2.wait()
    g3.wait()

    for g in range(_BPW // _L):
        sl = pl.ds(g * _L, _L)
        bs_v[sl] = fb_v[sl] + cb_v[sl]

    g0.wait()
    g1.wait()

    lane = lax.broadcasted_iota(jnp.int32, (_L,), 0)

    def gbody(g, carry):
        ep_vec = jnp.zeros((_L,), jnp.float32)
        fo_vec = (fi_v[pl.ds(g * _L, _L)] & 1) * D
        co_vec = (ci_v[pl.ds(g * _L, _L)] & 1) * D
        for j in range(_L):
            b = g * _L + j
            foff = fo_vec[j]
            coff = co_vec[j]
            acc = fe_v[b, pl.ds(foff, _L)] * ce_v[b, pl.ds(coff, _L)]
            for k in range(1, D // _L):
                acc = acc + (fe_v[b, pl.ds(foff + k * _L, _L)]
                             * ce_v[b, pl.ds(coff + k * _L, _L)])
            for sh in (8, 4, 2, 1):
                acc = acc + _lane_perm(acc, lane ^ sh)
            ep_vec = jnp.where(lane == j, acc, ep_vec)
        ep_v[pl.ds(g * _L, _L)] = ep_vec
        return carry

    lax.fori_loop(0, _BPW // _L, gbody, 0)

    pltpu.sync_copy(ep_v, ep_hbm.at[pl.ds(base, _BPW)])
    pltpu.sync_copy(bs_v, bs_hbm.at[pl.ds(base, _BPW)])


def _tc_finish(c_ref, ep_ref, bs_ref, o_ref):
    c = c_ref[...]
    lc = jnp.log(c)
    w = jnp.minimum(jnp.exp(lc * ALPHA) * (X_MAX ** -ALPHA), 1.0)
    a = ep_ref[...] - lc
    bs = bs_ref[...]
    s_wa2 = jnp.sum(w * a * a)
    s_wa = jnp.sum(w * a)
    s_w = jnp.sum(w)
    s_b = jnp.sum(bs)
    s_b2 = jnp.sum(bs * bs)
    val = (B * s_wa2 + 2.0 * s_wa * s_b + s_w * s_b2) / (B * B)
    o_ref[...] = jnp.broadcast_to(val, (1, 1))


def kernel(focal_emb, context_emb, focal_bias, context_bias,
           focal_input, context_input, coocurrence_count):
    mesh = plsc.VectorSubcoreMesh(core_axis_name="c", subcore_axis_name="s")
    sc_fn = functools.partial(
        pl.kernel,
        mesh=mesh,
        out_type=[
            jax.ShapeDtypeStruct((B,), jnp.float32),
            jax.ShapeDtypeStruct((B,), jnp.float32),
        ],
        scratch_types=[
            pltpu.VMEM((_BPW,), jnp.int32),
            pltpu.VMEM((_BPW,), jnp.int32),
            pltpu.VMEM((_BPW,), jnp.int32),
            pltpu.VMEM((_BPW,), jnp.int32),
            pltpu.VMEM((_BPW, 2 * D), jnp.float32),
            pltpu.VMEM((_BPW, 2 * D), jnp.float32),
            pltpu.VMEM((_BPW,), jnp.float32),
            pltpu.VMEM((_BPW,), jnp.float32),
            pltpu.VMEM((_BPW,), jnp.float32),
            pltpu.VMEM((_BPW,), jnp.float32),
            pltpu.SemaphoreType.DMA,
            pltpu.SemaphoreType.DMA,
            pltpu.SemaphoreType.DMA,
            pltpu.SemaphoreType.DMA,
        ],
    )(_sc_gather_dot)

    ep, bs = sc_fn(focal_emb.reshape(V // 2, 2 * D),
                   context_emb.reshape(V // 2, 2 * D),
                   focal_bias.reshape(V), context_bias.reshape(V),
                   focal_input, context_input)

    out = pl.pallas_call(
        _tc_finish,
        out_shape=jax.ShapeDtypeStruct((1, 1), jnp.float32),
    )(coocurrence_count.reshape(32, B // 32),
      ep.reshape(32, B // 32), bs.reshape(32, B // 32))
    return out.reshape(())


# speedup vs baseline: 1.4666x; 1.4666x over previous
import functools

import jax
import jax.numpy as jnp
from jax import lax
from jax.experimental import pallas as pl
from jax.experimental.pallas import tpu as pltpu
from jax.experimental.pallas import tpu_sc as plsc

V = 100000
D = 64
B = 4096
X_MAX = 100.0
ALPHA = 0.75

_NC = 2
_NS = 16
_NW = _NC * _NS
_BPW = B // _NW
_L = 16

_VBLK = 1024
_NBLK = 49
_H = _VBLK * _NBLK

_GATHER_DN = lax.GatherDimensionNumbers(
    offset_dims=(), collapsed_slice_dims=(0,), start_index_map=(0,))


def _lane_perm(x, idx):
    return lax.gather(x, idx.reshape(_L, 1), _GATHER_DN, (1,),
                      mode=lax.GatherScatterMode.PROMISE_IN_BOUNDS)


def _tc_transpose(ftl_ref, fth_ref, ctl_ref, cth_ref, fo_ref, co_ref):
    fo_ref[...] = jnp.concatenate(
        [jnp.transpose(ftl_ref[...]), jnp.transpose(fth_ref[...])], axis=1)
    co_ref[...] = jnp.concatenate(
        [jnp.transpose(ctl_ref[...]), jnp.transpose(cth_ref[...])], axis=1)


def _sc_gather_dot(fe_hbm, ce_hbm, fb_hbm, cb_hbm, fi_hbm, ci_hbm,
                   ep_hbm, bs_hbm,
                   fi_v, ci_v, fi2_v, ci2_v, fe_v, ce_v, fb_v, cb_v,
                   ep_v, bs_v,
                   sem0, sem1, sem2, sem3):
    wid = lax.axis_index("s") * _NC + lax.axis_index("c")
    base = wid * _BPW

    pltpu.sync_copy(fi_hbm.at[pl.ds(base, _BPW)], fi_v)
    pltpu.sync_copy(ci_hbm.at[pl.ds(base, _BPW)], ci_v)

    for g in range(_BPW // _L):
        sl = pl.ds(g * _L, _L)
        fh = jnp.where(fi_v[sl] >= _H, 1, 0)
        ch = jnp.where(ci_v[sl] >= _H, 1, 0)
        fi2_v[sl] = fi_v[sl] - fh * _H
        ci2_v[sl] = ci_v[sl] - ch * _H

    g0 = pltpu.async_copy(fe_hbm.at[fi2_v], fe_v, sem0)
    g1 = pltpu.async_copy(ce_hbm.at[ci2_v], ce_v, sem1)
    g2 = pltpu.async_copy(fb_hbm.at[fi_v], fb_v, sem2)
    g3 = pltpu.async_copy(cb_hbm.at[ci_v], cb_v, sem3)
    g2.wait()
    g3.wait()

    for g in range(_BPW // _L):
        sl = pl.ds(g * _L, _L)
        bs_v[sl] = fb_v[sl] + cb_v[sl]

    g0.wait()
    g1.wait()

    lane = lax.broadcasted_iota(jnp.int32, (_L,), 0)

    def gbody(g, carry):
        ep_vec = jnp.zeros((_L,), jnp.float32)
        fo_vec = jnp.where(fi_v[pl.ds(g * _L, _L)] >= _H, D, 0)
        co_vec = jnp.where(ci_v[pl.ds(g * _L, _L)] >= _H, D, 0)
        for j in range(_L):
            b = g * _L + j
            foff = fo_vec[j]
            coff = co_vec[j]
            acc = fe_v[b, pl.ds(foff, _L)] * ce_v[b, pl.ds(coff, _L)]
            for k in range(1, D // _L):
                acc = acc + (fe_v[b, pl.ds(foff + k * _L, _L)]
                             * ce_v[b, pl.ds(coff + k * _L, _L)])
            for sh in (8, 4, 2, 1):
                acc = acc + _lane_perm(acc, lane ^ sh)
            ep_vec = jnp.where(lane == j, acc, ep_vec)
        ep_v[pl.ds(g * _L, _L)] = ep_vec
        return carry

    lax.fori_loop(0, _BPW // _L, gbody, 0)

    pltpu.sync_copy(ep_v, ep_hbm.at[pl.ds(base, _BPW)])
    pltpu.sync_copy(bs_v, bs_hbm.at[pl.ds(base, _BPW)])


def _tc_finish(c_ref, ep_ref, bs_ref, o_ref):
    c = c_ref[...]
    lc = jnp.log(c)
    w = jnp.minimum(jnp.exp(lc * ALPHA) * (X_MAX ** -ALPHA), 1.0)
    a = ep_ref[...] - lc
    bs = bs_ref[...]
    s_wa2 = jnp.sum(w * a * a)
    s_wa = jnp.sum(w * a)
    s_w = jnp.sum(w)
    s_b = jnp.sum(bs)
    s_b2 = jnp.sum(bs * bs)
    val = (B * s_wa2 + 2.0 * s_wa * s_b + s_w * s_b2) / (B * B)
    o_ref[...] = jnp.broadcast_to(val, (1, 1))


def kernel(focal_emb, context_emb, focal_bias, context_bias,
           focal_input, context_input, coocurrence_count):
    fe2, ce2 = pl.pallas_call(
        _tc_transpose,
        grid=(_NBLK,),
        in_specs=[
            pl.BlockSpec((D, _VBLK), lambda i: (0, i)),
            pl.BlockSpec((D, _VBLK), lambda i: (0, i + _NBLK)),
            pl.BlockSpec((D, _VBLK), lambda i: (0, i)),
            pl.BlockSpec((D, _VBLK), lambda i: (0, i + _NBLK)),
        ],
        out_specs=[
            pl.BlockSpec((_VBLK, 2 * D), lambda i: (i, 0)),
            pl.BlockSpec((_VBLK, 2 * D), lambda i: (i, 0)),
        ],
        out_shape=[
            jax.ShapeDtypeStruct((_H, 2 * D), jnp.float32),
            jax.ShapeDtypeStruct((_H, 2 * D), jnp.float32),
        ],
    )(focal_emb.T, focal_emb.T, context_emb.T, context_emb.T)

    mesh = plsc.VectorSubcoreMesh(core_axis_name="c", subcore_axis_name="s")
    sc_fn = functools.partial(
        pl.kernel,
        mesh=mesh,
        out_type=[
            jax.ShapeDtypeStruct((B,), jnp.float32),
            jax.ShapeDtypeStruct((B,), jnp.float32),
        ],
        scratch_types=[
            pltpu.VMEM((_BPW,), jnp.int32),
            pltpu.VMEM((_BPW,), jnp.int32),
            pltpu.VMEM((_BPW,), jnp.int32),
            pltpu.VMEM((_BPW,), jnp.int32),
            pltpu.VMEM((_BPW, 2 * D), jnp.float32),
            pltpu.VMEM((_BPW, 2 * D), jnp.float32),
            pltpu.VMEM((_BPW,), jnp.float32),
            pltpu.VMEM((_BPW,), jnp.float32),
            pltpu.VMEM((_BPW,), jnp.float32),
            pltpu.VMEM((_BPW,), jnp.float32),
            pltpu.SemaphoreType.DMA,
            pltpu.SemaphoreType.DMA,
            pltpu.SemaphoreType.DMA,
            pltpu.SemaphoreType.DMA,
        ],
    )(_sc_gather_dot)

    ep, bs = sc_fn(fe2, ce2,
                   focal_bias.reshape(V), context_bias.reshape(V),
                   focal_input, context_input)

    out = pl.pallas_call(
        _tc_finish,
        out_shape=jax.ShapeDtypeStruct((1, 1), jnp.float32),
    )(coocurrence_count.reshape(32, B // 32),
      ep.reshape(32, B // 32), bs.reshape(32, B // 32))
    return out.reshape(())
